# baseline (speedup 1.0000x reference)
import functools

import jax
import jax.numpy as jnp
from jax import lax
from jax.experimental import pallas as pl
from jax.experimental.pallas import tpu as pltpu
from jax.experimental.pallas import tpu_sc as plsc

M_TILE = 1024
K_CHUNK = 2736


def _argmin_body(x_ref, t_ref, e_ref, v0_ref, v1_ref, v2_ref, io_ref,
                 ind_ref):
    x = x_ref[...]
    t = t_ref[...]
    k_total = e_ref.shape[0]
    acc = None
    best = None
    for c, v_ref in enumerate((v0_ref, v1_ref, v2_ref)):
        lo = c * K_CHUNK
        wc = min(K_CHUNK, k_total - lo)
        e_c = e_ref[pl.ds(lo, wc), :]
        s2 = lax.dot_general(x, e_c, (((1,), (1,)), ((), ())),
                             preferred_element_type=jnp.float32)
        d_c = (t - s2) + v_ref[...]
        mn = jnp.min(d_c, axis=1, keepdims=True)
        io = io_ref[:, :wc]
        idxf = jnp.min(jnp.where(d_c == mn, io, jnp.float32(1e9)),
                       axis=1, keepdims=True)
        idx = idxf.astype(jnp.int32) + lo
        if acc is None:
            acc = mn.astype(jnp.bfloat16).astype(jnp.float32)
            best = idx
        else:
            win = mn < acc
            acc = jnp.where(win, mn.astype(jnp.bfloat16).astype(jnp.float32),
                            acc)
            best = jnp.where(win, idx, best)
    ind_ref[...] = best


def _nearest_code(flat, t, embed, v0, v1, v2):
    m, d = flat.shape
    k = embed.shape[0]
    grid = (m // M_TILE,)
    iof = jnp.arange(K_CHUNK, dtype=jnp.float32)[None, :]
    return pl.pallas_call(
        _argmin_body,
        grid=grid,
        in_specs=[
            pl.BlockSpec((M_TILE, d), lambda i: (i, 0)),
            pl.BlockSpec((M_TILE, 1), lambda i: (i, 0)),
            pl.BlockSpec((k, d), lambda i: (0, 0)),
            pl.BlockSpec(v0.shape, lambda i: (0, 0)),
            pl.BlockSpec(v1.shape, lambda i: (0, 0)),
            pl.BlockSpec(v2.shape, lambda i: (0, 0)),
            pl.BlockSpec(iof.shape, lambda i: (0, 0)),
        ],
        out_specs=pl.BlockSpec((M_TILE, 1), lambda i: (i, 0)),
        out_shape=jax.ShapeDtypeStruct((m, 1), jnp.int32),
    )(flat, t, embed, v0, v1, v2, iof)


def _make_sc_gather(k, d, m):
    info = plsc.get_sparse_core_info()
    nc, ns = info.num_cores, info.num_subcores
    nw = nc * ns
    b_per_w = m // nw
    chunk = 128
    n_chunks = b_per_w // chunk
    mesh = plsc.VectorSubcoreMesh(core_axis_name="c", subcore_axis_name="s")

    @functools.partial(
        pl.kernel, mesh=mesh,
        out_type=jax.ShapeDtypeStruct((m, d), jnp.float32),
        scratch_types=[
            pltpu.VMEM((n_chunks, chunk), jnp.int32),
            pltpu.VMEM((chunk, d), jnp.float32),
            pltpu.SemaphoreType.DMA,
        ],
    )
    def gather_k(table_hbm, idx_hbm, out_hbm, idx_v, rows_v, sem):
        wid = lax.axis_index("s") * nc + lax.axis_index("c")
        base = wid * b_per_w
        pltpu.sync_copy(idx_hbm.at[wid], idx_v)
        for c in range(n_chunks):
            pltpu.async_copy(table_hbm.at[idx_v.at[c]], rows_v, sem).wait()
            pltpu.sync_copy(rows_v, out_hbm.at[pl.ds(base + c * chunk, chunk)])

    def run(embed, idx_flat):
        idx3 = idx_flat.reshape(nw, n_chunks, chunk)
        return gather_k(embed, idx3)

    return run


def kernel(z, embed):
    b, n, d = z.shape
    k = embed.shape[0]
    m = b * n
    flat = z.reshape(-1, d)
    t = jnp.sum(flat * flat, axis=1, keepdims=True)
    flat2 = flat * 2.0
    v = jnp.sum(embed * embed, axis=1)[None, :]
    v0 = v[:, :K_CHUNK]
    v1 = v[:, K_CHUNK:2 * K_CHUNK]
    v2 = v[:, 2 * K_CHUNK:]
    ind = _nearest_code(flat2, t, embed, v0, v1, v2).reshape(-1)
    quantize = _make_sc_gather(k, d, m)(embed, ind)
    return quantize.reshape(b, n, d), ind.reshape(b, n)


# speedup vs baseline: 1.0149x; 1.0149x over previous
import functools

import jax
import jax.numpy as jnp
from jax import lax
from jax.experimental import pallas as pl
from jax.experimental.pallas import tpu as pltpu
from jax.experimental.pallas import tpu_sc as plsc

M_TILE = 1024
K_CHUNK = 2736


def _argmin_body(x_ref, t_ref, e_ref, v0_ref, v1_ref, v2_ref, io_ref,
                 ind_ref):
    x = x_ref[...] * 2.0
    t = t_ref[...]
    k_total = e_ref.shape[0]
    acc = None
    best = None
    for c, v_ref in enumerate((v0_ref, v1_ref, v2_ref)):
        lo = c * K_CHUNK
        wc = min(K_CHUNK, k_total - lo)
        e_c = e_ref[pl.ds(lo, wc), :]
        s2 = lax.dot_general(x, e_c, (((1,), (1,)), ((), ())),
                             preferred_element_type=jnp.float32)
        d_c = (t - s2) + v_ref[...]
        mn = jnp.min(d_c, axis=1, keepdims=True)
        io = io_ref[:, :wc]
        idxf = jnp.min(jnp.where(d_c == mn, io, jnp.float32(1e9)),
                       axis=1, keepdims=True)
        idx = idxf.astype(jnp.int32) + lo
        if acc is None:
            acc = mn.astype(jnp.bfloat16).astype(jnp.float32)
            best = idx
        else:
            win = mn < acc
            acc = jnp.where(win, mn.astype(jnp.bfloat16).astype(jnp.float32),
                            acc)
            best = jnp.where(win, idx, best)
    ind_ref[...] = best


def _nearest_code(flat, t, embed, v0, v1, v2):
    m, d = flat.shape
    k = embed.shape[0]
    grid = (m // M_TILE,)
    iof = jnp.arange(K_CHUNK, dtype=jnp.float32)[None, :]
    return pl.pallas_call(
        _argmin_body,
        grid=grid,
        in_specs=[
            pl.BlockSpec((M_TILE, d), lambda i: (i, 0)),
            pl.BlockSpec((M_TILE, 1), lambda i: (i, 0)),
            pl.BlockSpec((k, d), lambda i: (0, 0)),
            pl.BlockSpec(v0.shape, lambda i: (0, 0)),
            pl.BlockSpec(v1.shape, lambda i: (0, 0)),
            pl.BlockSpec(v2.shape, lambda i: (0, 0)),
            pl.BlockSpec(iof.shape, lambda i: (0, 0)),
        ],
        out_specs=pl.BlockSpec((M_TILE, 1), lambda i: (i, 0)),
        out_shape=jax.ShapeDtypeStruct((m, 1), jnp.int32),
    )(flat, t, embed, v0, v1, v2, iof)


def _make_sc_gather(k, d, m):
    info = plsc.get_sparse_core_info()
    nc, ns = info.num_cores, info.num_subcores
    nw = nc * ns
    b_per_w = m // nw
    chunk = 128
    n_chunks = b_per_w // chunk
    mesh = plsc.VectorSubcoreMesh(core_axis_name="c", subcore_axis_name="s")

    @functools.partial(
        pl.kernel, mesh=mesh,
        out_type=jax.ShapeDtypeStruct((m, d), jnp.float32),
        scratch_types=[
            pltpu.VMEM((n_chunks, chunk), jnp.int32),
            pltpu.VMEM((chunk, d), jnp.float32),
            pltpu.SemaphoreType.DMA,
        ],
    )
    def gather_k(table_hbm, idx_hbm, out_hbm, idx_v, rows_v, sem):
        wid = lax.axis_index("s") * nc + lax.axis_index("c")
        base = wid * b_per_w
        pltpu.sync_copy(idx_hbm.at[wid], idx_v)
        for c in range(n_chunks):
            pltpu.async_copy(table_hbm.at[idx_v.at[c]], rows_v, sem).wait()
            pltpu.sync_copy(rows_v, out_hbm.at[pl.ds(base + c * chunk, chunk)])

    def run(embed, idx_flat):
        idx3 = idx_flat.reshape(nw, n_chunks, chunk)
        return gather_k(embed, idx3)

    return run


def kernel(z, embed):
    b, n, d = z.shape
    k = embed.shape[0]
    m = b * n
    flat = z.reshape(-1, d)
    t = jnp.sum(flat * flat, axis=1, keepdims=True)
    v = jnp.sum(embed * embed, axis=1)[None, :]
    v0 = v[:, :K_CHUNK]
    v1 = v[:, K_CHUNK:2 * K_CHUNK]
    v2 = v[:, 2 * K_CHUNK:]
    ind = _nearest_code(flat, t, embed, v0, v1, v2).reshape(-1)
    quantize = _make_sc_gather(k, d, m)(embed, ind)
    return quantize.reshape(b, n, d), ind.reshape(b, n)


# speedup vs baseline: 1.0163x; 1.0014x over previous
import functools

import jax
import jax.numpy as jnp
from jax import lax
from jax.experimental import pallas as pl
from jax.experimental.pallas import tpu as pltpu
from jax.experimental.pallas import tpu_sc as plsc

M_TILE = 1024
K_CHUNK = 2736


def _argmin_body(x_ref, t_ref, e_ref, v0_ref, v1_ref, v2_ref, io_ref,
                 ind_ref):
    x = x_ref[...]
    t = t_ref[...]
    k_total = e_ref.shape[0]
    acc = None
    best = None
    for c, v_ref in enumerate((v0_ref, v1_ref, v2_ref)):
        lo = c * K_CHUNK
        wc = min(K_CHUNK, k_total - lo)
        e_c = e_ref[pl.ds(lo, wc), :]
        s2 = 2.0 * lax.dot_general(x, e_c, (((1,), (1,)), ((), ())),
                                   preferred_element_type=jnp.float32)
        d_c = (t - s2) + v_ref[...]
        mn = jnp.min(d_c, axis=1, keepdims=True)
        io = io_ref[:, :wc]
        idxf = jnp.min(jnp.where(d_c == mn, io, jnp.float32(1e9)),
                       axis=1, keepdims=True)
        idx = idxf.astype(jnp.int32) + lo
        if acc is None:
            acc = mn.astype(jnp.bfloat16).astype(jnp.float32)
            best = idx
        else:
            win = mn < acc
            acc = jnp.where(win, mn.astype(jnp.bfloat16).astype(jnp.float32),
                            acc)
            best = jnp.where(win, idx, best)
    ind_ref[...] = best


def _nearest_code(flat, t, embed, v0, v1, v2):
    m, d = flat.shape
    k = embed.shape[0]
    grid = (m // M_TILE,)
    iof = jnp.arange(K_CHUNK, dtype=jnp.float32)[None, :]
    return pl.pallas_call(
        _argmin_body,
        grid=grid,
        in_specs=[
            pl.BlockSpec((M_TILE, d), lambda i: (i, 0)),
            pl.BlockSpec((M_TILE, 1), lambda i: (i, 0)),
            pl.BlockSpec((k, d), lambda i: (0, 0)),
            pl.BlockSpec(v0.shape, lambda i: (0, 0)),
            pl.BlockSpec(v1.shape, lambda i: (0, 0)),
            pl.BlockSpec(v2.shape, lambda i: (0, 0)),
            pl.BlockSpec(iof.shape, lambda i: (0, 0)),
        ],
        out_specs=pl.BlockSpec((M_TILE, 1), lambda i: (i, 0)),
        out_shape=jax.ShapeDtypeStruct((m, 1), jnp.int32),
    )(flat, t, embed, v0, v1, v2, iof)


def _make_sc_gather(k, d, m):
    info = plsc.get_sparse_core_info()
    nc, ns = info.num_cores, info.num_subcores
    nw = nc * ns
    b_per_w = m // nw
    chunk = 128
    n_chunks = b_per_w // chunk
    mesh = plsc.VectorSubcoreMesh(core_axis_name="c", subcore_axis_name="s")

    @functools.partial(
        pl.kernel, mesh=mesh,
        out_type=jax.ShapeDtypeStruct((m, d), jnp.float32),
        scratch_types=[
            pltpu.VMEM((n_chunks, chunk), jnp.int32),
            pltpu.VMEM((chunk, d), jnp.float32),
            pltpu.SemaphoreType.DMA,
        ],
    )
    def gather_k(table_hbm, idx_hbm, out_hbm, idx_v, rows_v, sem):
        wid = lax.axis_index("s") * nc + lax.axis_index("c")
        base = wid * b_per_w
        pltpu.sync_copy(idx_hbm.at[wid], idx_v)
        for c in range(n_chunks):
            pltpu.async_copy(table_hbm.at[idx_v.at[c]], rows_v, sem).wait()
            pltpu.sync_copy(rows_v, out_hbm.at[pl.ds(base + c * chunk, chunk)])

    def run(embed, idx_flat):
        idx3 = idx_flat.reshape(nw, n_chunks, chunk)
        return gather_k(embed, idx3)

    return run


def kernel(z, embed):
    b, n, d = z.shape
    k = embed.shape[0]
    m = b * n
    flat = z.reshape(-1, d)
    t = jnp.sum(flat * flat, axis=1, keepdims=True)
    v = jnp.sum(embed * embed, axis=1)[None, :]
    v0 = v[:, :K_CHUNK]
    v1 = v[:, K_CHUNK:2 * K_CHUNK]
    v2 = v[:, 2 * K_CHUNK:]
    embed_bf = embed.astype(jnp.bfloat16)
    ind = _nearest_code(flat, t, embed_bf, v0, v1, v2).reshape(-1)
    quantize = _make_sc_gather(k, d, m)(embed, ind)
    return quantize.reshape(b, n, d), ind.reshape(b, n)


# speedup vs baseline: 1.0357x; 1.0191x over previous
import functools

import jax
import jax.numpy as jnp
from jax import lax
from jax.experimental import pallas as pl
from jax.experimental.pallas import tpu as pltpu
from jax.experimental.pallas import tpu_sc as plsc

M_TILE = 1024
K_CHUNK = 2736


def _argmin_body(x_ref, t_ref, e_ref, v0_ref, v1_ref, v2_ref, io_ref,
                 ind_ref):
    x = x_ref[...]
    t = t_ref[...]
    k_total = e_ref.shape[0]
    acc = None
    best = None
    for c, v_ref in enumerate((v0_ref, v1_ref, v2_ref)):
        lo = c * K_CHUNK
        wc = min(K_CHUNK, k_total - lo)
        e_c = e_ref[pl.ds(lo, wc), :]
        s2 = 2.0 * lax.dot_general(x, e_c, (((1,), (1,)), ((), ())),
                                   preferred_element_type=jnp.float32)
        d_c = (t - s2) + v_ref[...]
        mn = jnp.min(d_c, axis=1, keepdims=True)
        io = io_ref[:, :wc]
        idxf = jnp.min(jnp.where(d_c == mn, io, jnp.float32(1e9)),
                       axis=1, keepdims=True)
        idx = idxf.astype(jnp.int32) + lo
        if acc is None:
            acc = mn.astype(jnp.bfloat16).astype(jnp.float32)
            best = idx
        else:
            win = mn < acc
            acc = jnp.where(win, mn.astype(jnp.bfloat16).astype(jnp.float32),
                            acc)
            best = jnp.where(win, idx, best)
    ind_ref[...] = best


def _nearest_code(flat, t, embed, v0, v1, v2):
    m, d = flat.shape
    k = embed.shape[0]
    grid = (m // M_TILE,)
    iof = jnp.arange(K_CHUNK, dtype=jnp.float32)[None, :]
    return pl.pallas_call(
        _argmin_body,
        grid=grid,
        in_specs=[
            pl.BlockSpec((M_TILE, d), lambda i: (i, 0)),
            pl.BlockSpec((M_TILE, 1), lambda i: (i, 0)),
            pl.BlockSpec((k, d), lambda i: (0, 0)),
            pl.BlockSpec(v0.shape, lambda i: (0, 0)),
            pl.BlockSpec(v1.shape, lambda i: (0, 0)),
            pl.BlockSpec(v2.shape, lambda i: (0, 0)),
            pl.BlockSpec(iof.shape, lambda i: (0, 0)),
        ],
        out_specs=pl.BlockSpec((M_TILE, 1), lambda i: (i, 0)),
        out_shape=jax.ShapeDtypeStruct((m, 1), jnp.int32),
    )(flat, t, embed, v0, v1, v2, iof)


def _make_sc_gather(k, d, m):
    info = plsc.get_sparse_core_info()
    nc, ns = info.num_cores, info.num_subcores
    nw = nc * ns
    b_per_w = m // nw
    chunk = 128
    n_chunks = b_per_w // chunk
    mesh = plsc.VectorSubcoreMesh(core_axis_name="c", subcore_axis_name="s")

    @functools.partial(
        pl.kernel, mesh=mesh,
        out_type=jax.ShapeDtypeStruct((m, d), jnp.float32),
        scratch_types=[
            pltpu.VMEM((n_chunks, chunk), jnp.int32),
            pltpu.VMEM((chunk, d), jnp.float32),
            pltpu.SemaphoreType.DMA,
        ],
    )
    def gather_k(table_hbm, idx_hbm, out_hbm, idx_v, rows_v, sem):
        wid = lax.axis_index("s") * nc + lax.axis_index("c")
        base = wid * b_per_w
        pltpu.sync_copy(idx_hbm.at[wid], idx_v)
        for c in range(n_chunks):
            pltpu.async_copy(table_hbm.at[idx_v.at[c]], rows_v, sem).wait()
            pltpu.sync_copy(rows_v, out_hbm.at[pl.ds(base + c * chunk, chunk)])

    def run(embed, idx_flat):
        idx3 = idx_flat.reshape(nw, n_chunks, chunk)
        return gather_k(embed, idx3)

    return run


def kernel(z, embed):
    b, n, d = z.shape
    k = embed.shape[0]
    m = b * n
    flat = z.reshape(-1, d)
    t = jnp.sum(flat * flat, axis=1, keepdims=True)
    v = jnp.sum(embed * embed, axis=1)[None, :]
    v0 = v[:, :K_CHUNK]
    v1 = v[:, K_CHUNK:2 * K_CHUNK]
    v2 = v[:, 2 * K_CHUNK:]
    ind = _nearest_code(flat, t, embed, v0, v1, v2).reshape(-1)
    quantize = _make_sc_gather(k, d, m)(embed, ind)
    return quantize.reshape(b, n, d), ind.reshape(b, n)
